# Initial kernel scaffold; baseline (speedup 1.0000x reference)
import functools

import jax
import jax.numpy as jnp
from jax import lax
from jax.experimental import pallas as pl
from jax.experimental.pallas import tpu as pltpu
from jax.experimental.pallas import tpu_sc as plsc

N = 10000
E = 320000
D = 128
H = 64

NC = 2
NS = 16
NW = NC * NS
EW = E // NW
C = 80
NCH = EW // C
RPS = N // NS


def _mesh():
    return plsc.VectorSubcoreMesh(
        core_axis_name="c", subcore_axis_name="s", num_cores=NC, num_subcores=NS
    )


def _tc_first(x, W1):
    def body(xr, wr, outr):
        outr[...] = jnp.dot(xr[...], wr[...], preferred_element_type=jnp.float32)

    return pl.pallas_call(
        body, out_shape=jax.ShapeDtypeStruct((N, H), jnp.float32)
    )(x, W1)


def _tc_mid(p, b, W):
    def body(pr, br, wr, outr):
        hrelu = jnp.maximum(pr[0] + pr[1] + br[...], 0.0)
        outr[...] = jnp.dot(hrelu, wr[...], preferred_element_type=jnp.float32)

    return pl.pallas_call(
        body, out_shape=jax.ShapeDtypeStruct((N, H), jnp.float32)
    )(p, b, W)


def _tc_head(p, b, Wout):
    def body(pr, br, wr, outr):
        hrelu = jnp.maximum(pr[0] + pr[1] + br[...], 0.0)
        outr[...] = jnp.dot(hrelu, wr[...], preferred_element_type=jnp.float32)

    return pl.pallas_call(
        body, out_shape=jax.ShapeDtypeStruct((N, 1), jnp.float32)
    )(p, b, Wout)


def _tc_final(q, bout):
    def body(qr, br, outr):
        outr[...] = jax.nn.softplus(qr[0] + qr[1] + br[...])

    return pl.pallas_call(
        body, out_shape=jax.ShapeDtypeStruct((N, 1), jnp.float32)
    )(q, bout)


def _spmm64(h, srcw, dstw, aw, zeros):
    @functools.partial(
        pl.kernel,
        out_type=jax.ShapeDtypeStruct((NC, N, H), jnp.float32),
        mesh=_mesh(),
        scratch_types=[
            pltpu.VMEM((NCH, C), jnp.int32),
            pltpu.VMEM((NCH, C), jnp.int32),
            pltpu.VMEM((NCH, C), jnp.float32),
            pltpu.VMEM((C, H), jnp.float32),
            pltpu.VMEM_SHARED((N, H), jnp.float32),
            pltpu.SemaphoreType.DMA,
        ],
    )
    def k(h_hbm, src_hbm, dst_hbm, a_hbm, z_hbm, out_hbm,
          src_v, dst_v, a_v, rows_v, acc_sh, sem):
        cid = lax.axis_index("c")
        sid = lax.axis_index("s")
        wid = sid * NC + cid

        pltpu.sync_copy(z_hbm.at[pl.ds(sid * RPS, RPS)],
                        acc_sh.at[pl.ds(sid * RPS, RPS)])
        pltpu.sync_copy(src_hbm.at[wid], src_v)
        pltpu.sync_copy(dst_hbm.at[wid], dst_v)
        pltpu.sync_copy(a_hbm.at[wid], a_v)
        plsc.subcore_barrier()

        def chunk(j, carry):
            pltpu.async_copy(h_hbm.at[src_v.at[j]], rows_v, sem).wait()

            def blk(b, carry2):
                for t in range(16):
                    e = b * 16 + t
                    ji = jnp.full((16,), j, jnp.int32)
                    ei = jnp.full((16,), e, jnp.int32)
                    av = plsc.load_gather(a_v, [ji, ei])
                    for f in range(H // 16):
                        sl = (e, pl.ds(f * 16, 16))
                        rows_v[sl] = rows_v[sl] * av
                return carry2

            lax.fori_loop(0, C // 16, blk, 0)
            pltpu.sync_copy(rows_v, acc_sh.at[dst_v.at[j]], add=True)
            return carry

        lax.fori_loop(0, NCH, chunk, 0)
        plsc.subcore_barrier()
        pltpu.sync_copy(acc_sh.at[pl.ds(sid * RPS, RPS)],
                        out_hbm.at[cid, pl.ds(sid * RPS, RPS)])

    return k(h, srcw, dstw, aw, zeros)


def _spmm1(t2, srcw, dstw, aw, zeros1):
    @functools.partial(
        pl.kernel,
        out_type=jax.ShapeDtypeStruct((NC, N, 1), jnp.float32),
        mesh=_mesh(),
        scratch_types=[
            pltpu.VMEM((NCH, C), jnp.int32),
            pltpu.VMEM((NCH, C), jnp.int32),
            pltpu.VMEM((NCH, C), jnp.float32),
            pltpu.VMEM((N, 1), jnp.float32),
            pltpu.VMEM((C, 1), jnp.float32),
            pltpu.VMEM_SHARED((N, 1), jnp.float32),
            pltpu.SemaphoreType.DMA,
        ],
    )
    def k(t2_hbm, src_hbm, dst_hbm, a_hbm, z_hbm, out_hbm,
          src_v, dst_v, a_v, t2_v, vals_v, acc_sh, sem):
        cid = lax.axis_index("c")
        sid = lax.axis_index("s")
        wid = sid * NC + cid

        pltpu.sync_copy(z_hbm.at[pl.ds(sid * RPS, RPS)],
                        acc_sh.at[pl.ds(sid * RPS, RPS)])
        pltpu.sync_copy(src_hbm.at[wid], src_v)
        pltpu.sync_copy(dst_hbm.at[wid], dst_v)
        pltpu.sync_copy(a_hbm.at[wid], a_v)
        pltpu.sync_copy(t2_hbm, t2_v)
        plsc.subcore_barrier()

        zeros16 = jnp.zeros((16,), jnp.int32)
        lane = lax.iota(jnp.int32, 16)

        def chunk(j, carry):
            def blk(b, carry2):
                sidx = src_v[j, pl.ds(b * 16, 16)]
                av = a_v[j, pl.ds(b * 16, 16)]
                sv = plsc.load_gather(t2_v, [sidx, zeros16])
                plsc.store_scatter(vals_v, [b * 16 + lane, zeros16], av * sv)
                return carry2

            lax.fori_loop(0, C // 16, blk, 0)
            pltpu.sync_copy(vals_v, acc_sh.at[dst_v.at[j]], add=True)
            return carry

        lax.fori_loop(0, NCH, chunk, 0)
        plsc.subcore_barrier()
        pltpu.sync_copy(acc_sh.at[pl.ds(sid * RPS, RPS)],
                        out_hbm.at[cid, pl.ds(sid * RPS, RPS)])

    return k(t2, srcw, dstw, aw, zeros1)


def kernel(x, edge_index, adj_values, W1, b1, W2, b2, Wout, bout):
    dstw = edge_index[0].reshape(NW, NCH, C)
    srcw = edge_index[1].reshape(NW, NCH, C)
    aw = adj_values.reshape(NW, NCH, C)
    zeros = jnp.zeros((N, H), jnp.float32)
    zeros1 = jnp.zeros((N, 1), jnp.float32)

    t0 = _tc_first(x, W1)
    p1 = _spmm64(t0, srcw, dstw, aw, zeros)
    t1 = _tc_mid(p1, b1, W2)
    p2 = _spmm64(t1, srcw, dstw, aw, zeros)
    t2 = _tc_head(p2, b2, Wout)
    q = _spmm1(t2, srcw, dstw, aw, zeros1)
    return _tc_final(q, bout)


# speedup vs baseline: 6.7220x; 6.7220x over previous
import functools

import jax
import jax.numpy as jnp
from jax import lax
from jax.experimental import pallas as pl
from jax.experimental.pallas import tpu as pltpu
from jax.experimental.pallas import tpu_sc as plsc

N = 10000
E = 320000
D = 128
H = 64

NC = 2
NS = 16
NW = NC * NS
EW = E // NW
C = 80
NCH = EW // C
NP = 10240
RPS = NP // NS


_BCAST_DNUMS = lax.GatherDimensionNumbers(
    offset_dims=(), collapsed_slice_dims=(0,), start_index_map=(0,)
)


def _bcast(v, t):
    idx = jnp.full((16, 1), t, jnp.int32)
    return lax.gather(v, idx, _BCAST_DNUMS, (1,),
                      mode=lax.GatherScatterMode.PROMISE_IN_BOUNDS)


def _mesh():
    return plsc.VectorSubcoreMesh(
        core_axis_name="c", subcore_axis_name="s", num_cores=NC, num_subcores=NS
    )


def _tc_layer(p, W, b):
    def body(pr, wr, br, outr):
        hsum = pr[0][:N] + pr[1][:N]
        outr[...] = jnp.maximum(jnp.dot(hsum, wr[...]) + br[...], 0.0)

    return pl.pallas_call(
        body, out_shape=jax.ShapeDtypeStruct((N, H), jnp.float32)
    )(p, W, b)


def _tc_head(p, Wout_pad, bout):
    def body(pr, wr, br, outr):
        hsum = pr[0][:N] + pr[1][:N]
        z = jnp.dot(hsum, wr[...])[:, 0:1] + br[...]
        outr[...] = jax.nn.softplus(z)

    return pl.pallas_call(
        body, out_shape=jax.ShapeDtypeStruct((N, 1), jnp.float32)
    )(p, Wout_pad, bout)


def _spmm(h, srcw, dstw, aw, zeros, F, nbuf):
    @functools.partial(
        pl.kernel,
        out_type=jax.ShapeDtypeStruct((NC, NP, F), jnp.float32),
        mesh=_mesh(),
        compiler_params=pltpu.CompilerParams(use_tc_tiling_on_sc=False),
        scratch_types=[
            pltpu.VMEM((NCH, C), jnp.int32),
            pltpu.VMEM((NCH, C), jnp.int32),
            pltpu.VMEM((NCH, C), jnp.float32),
            pltpu.VMEM((nbuf, C, F), jnp.float32),
            pltpu.VMEM_SHARED((NP, F), jnp.float32),
            pltpu.SemaphoreType.DMA((nbuf,)),
        ],
    )
    def k(h_hbm, src_hbm, dst_hbm, a_hbm, z_hbm, out_hbm,
          src_v, dst_v, a_v, rows_v, acc_sh, sem):
        cid = lax.axis_index("c")
        sid = lax.axis_index("s")
        wid = sid * NC + cid

        pltpu.sync_copy(z_hbm.at[pl.ds(sid * RPS, RPS)],
                        acc_sh.at[pl.ds(sid * RPS, RPS)])
        pltpu.sync_copy(src_hbm.at[wid], src_v)
        pltpu.sync_copy(dst_hbm.at[wid], dst_v)
        pltpu.sync_copy(a_hbm.at[wid], a_v)
        plsc.subcore_barrier()

        if nbuf == 2:
            pltpu.async_copy(h_hbm.at[src_v.at[0]], rows_v.at[0], sem.at[0])

        def do_chunk(j, slot, nslot):
            if nbuf == 2:
                pltpu.make_async_copy(
                    h_hbm.at[src_v.at[j]], rows_v.at[slot], sem.at[slot]
                ).wait()

                @pl.when(j < NCH - 1)
                def _():
                    pltpu.async_copy(
                        h_hbm.at[src_v.at[j + 1]], rows_v.at[nslot],
                        sem.at[nslot]
                    )
            else:
                pltpu.async_copy(
                    h_hbm.at[src_v.at[j]], rows_v.at[slot], sem.at[slot]
                ).wait()

            def blk(b, carry2):
                av16 = a_v[j, pl.ds(b * 16, 16)]
                for t in range(16):
                    e = b * 16 + t
                    av = _bcast(av16, t)
                    for f in range(F // 16):
                        sl = (slot, e, pl.ds(f * 16, 16))
                        rows_v[sl] = rows_v[sl] * av
                return carry2

            lax.fori_loop(0, C // 16, blk, 0)
            pltpu.sync_copy(rows_v.at[slot], acc_sh.at[dst_v.at[j]], add=True)

        def chunk(j, carry):
            if nbuf == 2:
                parity = lax.rem(j, 2)

                @pl.when(parity == 0)
                def _():
                    do_chunk(j, 0, 1)

                @pl.when(parity == 1)
                def _():
                    do_chunk(j, 1, 0)
            else:
                do_chunk(j, 0, 0)

            return carry

        lax.fori_loop(0, NCH, chunk, 0)
        plsc.subcore_barrier()
        pltpu.sync_copy(acc_sh.at[pl.ds(sid * RPS, RPS)],
                        out_hbm.at[cid, pl.ds(sid * RPS, RPS)])

    return k(h, srcw, dstw, aw, zeros)


def kernel(x, edge_index, adj_values, W1, b1, W2, b2, Wout, bout):
    dstw = edge_index[0].reshape(NW, NCH, C)
    srcw = edge_index[1].reshape(NW, NCH, C)
    aw = adj_values.reshape(NW, NCH, C)
    zerosD = jnp.zeros((NP, D), jnp.float32)
    zerosH = jnp.zeros((NP, H), jnp.float32)
    Wout_pad = jnp.concatenate([Wout, jnp.zeros((H, 15), jnp.float32)], axis=1)

    p0 = _spmm(x, srcw, dstw, aw, zerosD, D, 1)
    h1 = _tc_layer(p0, W1, b1)
    p1 = _spmm(h1, srcw, dstw, aw, zerosH, H, 2)
    h2 = _tc_layer(p1, W2, b2)
    p2 = _spmm(h2, srcw, dstw, aw, zerosH, H, 2)
    return _tc_head(p2, Wout_pad, bout)


# speedup vs baseline: 7.9144x; 1.1774x over previous
import functools

import jax
import jax.numpy as jnp
from jax import lax
from jax.experimental import pallas as pl
from jax.experimental.pallas import tpu as pltpu
from jax.experimental.pallas import tpu_sc as plsc

N = 10000
E = 320000
D = 128
H = 64

NC = 2
NS = 16
NW = NC * NS
EW = E // NW
C = 80
NCH = EW // C
NP = 10240
RPS = NP // NS


_BCAST_DNUMS = lax.GatherDimensionNumbers(
    offset_dims=(), collapsed_slice_dims=(0,), start_index_map=(0,)
)


def _bcast(v, t):
    idx = jnp.full((16, 1), t, jnp.int32)
    return lax.gather(v, idx, _BCAST_DNUMS, (1,),
                      mode=lax.GatherScatterMode.PROMISE_IN_BOUNDS)


def _mesh():
    return plsc.VectorSubcoreMesh(
        core_axis_name="c", subcore_axis_name="s", num_cores=NC, num_subcores=NS
    )


def _tc_layer(p, W, b):
    def body(pr, wr, br, outr):
        hsum = pr[0][:N] + pr[1][:N]
        outr[...] = jnp.maximum(jnp.dot(hsum, wr[...]) + br[...], 0.0)

    return pl.pallas_call(
        body, out_shape=jax.ShapeDtypeStruct((N, H), jnp.float32)
    )(p, W, b)


def _tc_head(p, Wout_pad, bout):
    def body(pr, wr, br, outr):
        hsum = pr[0][:N] + pr[1][:N]
        z = jnp.dot(hsum, wr[...])[:, 0:1] + br[...]
        outr[...] = jax.nn.softplus(z)

    return pl.pallas_call(
        body, out_shape=jax.ShapeDtypeStruct((N, 1), jnp.float32)
    )(p, Wout_pad, bout)


def _spmm(h, srcw, dstw, aw, zeros, F):
    @functools.partial(
        pl.kernel,
        out_type=jax.ShapeDtypeStruct((NC, NP, F), jnp.float32),
        mesh=_mesh(),
        compiler_params=pltpu.CompilerParams(use_tc_tiling_on_sc=False),
        scratch_types=[
            pltpu.VMEM((2, C), jnp.int32),
            pltpu.VMEM((NCH, C), jnp.int32),
            pltpu.VMEM((NCH, C), jnp.float32),
            pltpu.VMEM((2, C, F), jnp.float32),
            pltpu.VMEM_SHARED((NP, F), jnp.float32),
            pltpu.SemaphoreType.DMA((2,)),
            pltpu.SemaphoreType.DMA((2,)),
        ],
    )
    def k(h_hbm, src_hbm, dst_hbm, a_hbm, z_hbm, out_hbm,
          srcb, dst_v, a_v, rows_v, acc_sh, rsem, ssem):
        cid = lax.axis_index("c")
        sid = lax.axis_index("s")
        wid = sid * NC + cid

        pltpu.sync_copy(z_hbm.at[pl.ds(sid * RPS, RPS)],
                        acc_sh.at[pl.ds(sid * RPS, RPS)])
        pltpu.sync_copy(dst_hbm.at[wid], dst_v)
        pltpu.sync_copy(a_hbm.at[wid], a_v)
        pltpu.sync_copy(src_hbm.at[wid, 0], srcb.at[0])
        plsc.subcore_barrier()

        pltpu.async_copy(h_hbm.at[srcb.at[0]], rows_v.at[0], rsem.at[0])
        pltpu.async_copy(src_hbm.at[wid, 1], srcb.at[1], ssem.at[1])

        def do_chunk(j, slot, nslot):
            @pl.when(j < NCH - 1)
            def _():
                pltpu.make_async_copy(
                    src_hbm.at[wid, j + 1], srcb.at[nslot], ssem.at[nslot]
                ).wait()
                pltpu.async_copy(
                    h_hbm.at[srcb.at[nslot]], rows_v.at[nslot], rsem.at[nslot]
                )

            pltpu.make_async_copy(
                h_hbm.at[srcb.at[slot]], rows_v.at[slot], rsem.at[slot]
            ).wait()

            @pl.when(j < NCH - 2)
            def _():
                pltpu.async_copy(
                    src_hbm.at[wid, j + 2], srcb.at[slot], ssem.at[slot]
                )

            def blk(b, carry2):
                av16 = a_v[j, pl.ds(b * 16, 16)]
                for t in range(16):
                    e = b * 16 + t
                    av = _bcast(av16, t)
                    for f in range(F // 16):
                        sl = (slot, e, pl.ds(f * 16, 16))
                        rows_v[sl] = rows_v[sl] * av
                return carry2

            lax.fori_loop(0, C // 16, blk, 0)
            pltpu.sync_copy(rows_v.at[slot], acc_sh.at[dst_v.at[j]], add=True)

        def chunk(j, carry):
            parity = lax.rem(j, 2)

            @pl.when(parity == 0)
            def _():
                do_chunk(j, 0, 1)

            @pl.when(parity == 1)
            def _():
                do_chunk(j, 1, 0)

            return carry

        lax.fori_loop(0, NCH, chunk, 0)
        plsc.subcore_barrier()
        pltpu.sync_copy(acc_sh.at[pl.ds(sid * RPS, RPS)],
                        out_hbm.at[cid, pl.ds(sid * RPS, RPS)])

    return k(h, srcw, dstw, aw, zeros)


def kernel(x, edge_index, adj_values, W1, b1, W2, b2, Wout, bout):
    dstw = edge_index[0].reshape(NW, NCH, C)
    srcw = edge_index[1].reshape(NW, NCH, C)
    aw = adj_values.reshape(NW, NCH, C)
    zerosD = jnp.zeros((NP, D), jnp.float32)
    zerosH = jnp.zeros((NP, H), jnp.float32)
    Wout_pad = jnp.concatenate([Wout, jnp.zeros((H, 15), jnp.float32)], axis=1)

    p0 = _spmm(x, srcw, dstw, aw, zerosD, D)
    h1 = _tc_layer(p0, W1, b1)
    p1 = _spmm(h1, srcw, dstw, aw, zerosH, H)
    h2 = _tc_layer(p1, W2, b2)
    p2 = _spmm(h2, srcw, dstw, aw, zerosH, H)
    return _tc_head(p2, Wout_pad, bout)


# speedup vs baseline: 10.4528x; 1.3207x over previous
import functools

import jax
import jax.numpy as jnp
from jax import lax
from jax.experimental import pallas as pl
from jax.experimental.pallas import tpu as pltpu
from jax.experimental.pallas import tpu_sc as plsc

N = 10000
E = 320000
D = 128
H = 64

NC = 2
NS = 16
NW = NC * NS
EW = E // NW
C = 80
NCH = EW // C
NP = 10240
RPS = NP // NS


_BCAST_DNUMS = lax.GatherDimensionNumbers(
    offset_dims=(), collapsed_slice_dims=(0,), start_index_map=(0,)
)


def _bcast(v, t):
    idx = jnp.full((16, 1), t, jnp.int32)
    return lax.gather(v, idx, _BCAST_DNUMS, (1,),
                      mode=lax.GatherScatterMode.PROMISE_IN_BOUNDS)


def _mesh():
    return plsc.VectorSubcoreMesh(
        core_axis_name="c", subcore_axis_name="s", num_cores=NC, num_subcores=NS
    )


def _tc_layer(p, W, b, K):
    def body(pr, wr, br, outr):
        hsum = pr[0][:N, :K] + pr[1][:N, :K]
        h = jnp.maximum(jnp.dot(hsum, wr[...]) + br[...], 0.0)
        outr[...] = jnp.concatenate(
            [h, jnp.zeros((N, D - H), jnp.float32)], axis=1)

    return pl.pallas_call(
        body, out_shape=jax.ShapeDtypeStruct((N, D), jnp.float32)
    )(p, W, b)


def _tc_head(p, Wout_pad, bout):
    def body(pr, wr, br, outr):
        hsum = pr[0][:N, :H] + pr[1][:N, :H]
        z = jnp.dot(hsum, wr[...])[:, 0:1] + br[...]
        outr[...] = jax.nn.softplus(z)

    return pl.pallas_call(
        body, out_shape=jax.ShapeDtypeStruct((N, 1), jnp.float32)
    )(p, Wout_pad, bout)


def _spmm(h, srcw, dstw, aw, zeros, F):
    @functools.partial(
        pl.kernel,
        out_type=jax.ShapeDtypeStruct((NC, NP, F), jnp.float32),
        mesh=_mesh(),
        compiler_params=pltpu.CompilerParams(use_tc_tiling_on_sc=False),
        scratch_types=[
            pltpu.VMEM((2, C), jnp.int32),
            pltpu.VMEM((NCH, C), jnp.int32),
            pltpu.VMEM((NCH, C), jnp.float32),
            pltpu.VMEM((2, C, F), jnp.float32),
            pltpu.VMEM_SHARED((NP, F), jnp.float32),
            pltpu.SemaphoreType.DMA((2,)),
            pltpu.SemaphoreType.DMA((2,)),
        ],
    )
    def k(h_hbm, src_hbm, dst_hbm, a_hbm, z_hbm, out_hbm,
          srcb, dst_v, a_v, rows_v, acc_sh, rsem, ssem):
        cid = lax.axis_index("c")
        sid = lax.axis_index("s")
        wid = sid * NC + cid

        pltpu.sync_copy(z_hbm.at[pl.ds(sid * RPS, RPS)],
                        acc_sh.at[pl.ds(sid * RPS, RPS)])
        pltpu.sync_copy(dst_hbm.at[wid], dst_v)
        pltpu.sync_copy(a_hbm.at[wid], a_v)
        pltpu.sync_copy(src_hbm.at[wid, 0], srcb.at[0])
        plsc.subcore_barrier()

        pltpu.async_copy(h_hbm.at[srcb.at[0]], rows_v.at[0], rsem.at[0])
        pltpu.async_copy(src_hbm.at[wid, 1], srcb.at[1], ssem.at[1])

        def do_chunk(j, slot, nslot):
            @pl.when(j < NCH - 1)
            def _():
                pltpu.make_async_copy(
                    src_hbm.at[wid, j + 1], srcb.at[nslot], ssem.at[nslot]
                ).wait()
                pltpu.async_copy(
                    h_hbm.at[srcb.at[nslot]], rows_v.at[nslot], rsem.at[nslot]
                )

            pltpu.make_async_copy(
                h_hbm.at[srcb.at[slot]], rows_v.at[slot], rsem.at[slot]
            ).wait()

            @pl.when(j < NCH - 2)
            def _():
                pltpu.async_copy(
                    src_hbm.at[wid, j + 2], srcb.at[slot], ssem.at[slot]
                )

            def blk(b, carry2):
                av16 = a_v[j, pl.ds(b * 16, 16)]
                for t in range(16):
                    e = b * 16 + t
                    av = _bcast(av16, t)
                    for f in range(F // 16):
                        sl = (slot, e, pl.ds(f * 16, 16))
                        rows_v[sl] = rows_v[sl] * av
                return carry2

            lax.fori_loop(0, C // 16, blk, 0)
            pltpu.sync_copy(rows_v.at[slot], acc_sh.at[dst_v.at[j]], add=True)

        def chunk(j, carry):
            parity = lax.rem(j, 2)

            @pl.when(parity == 0)
            def _():
                do_chunk(j, 0, 1)

            @pl.when(parity == 1)
            def _():
                do_chunk(j, 1, 0)

            return carry

        lax.fori_loop(0, NCH, chunk, 0)
        plsc.subcore_barrier()
        pltpu.sync_copy(acc_sh.at[pl.ds(sid * RPS, RPS)],
                        out_hbm.at[cid, pl.ds(sid * RPS, RPS)])

    return k(h, srcw, dstw, aw, zeros)


def kernel(x, edge_index, adj_values, W1, b1, W2, b2, Wout, bout):
    dstw = edge_index[0].reshape(NW, NCH, C)
    srcw = edge_index[1].reshape(NW, NCH, C)
    aw = adj_values.reshape(NW, NCH, C)
    zerosD = jnp.zeros((NP, D), jnp.float32)
    Wout_pad = jnp.concatenate([Wout, jnp.zeros((H, 15), jnp.float32)], axis=1)

    p0 = _spmm(x, srcw, dstw, aw, zerosD, D)
    h1 = _tc_layer(p0, W1, b1, D)
    p1 = _spmm(h1, srcw, dstw, aw, zerosD, D)
    h2 = _tc_layer(p1, W2, b2, H)
    p2 = _spmm(h2, srcw, dstw, aw, zerosD, D)
    return _tc_head(p2, Wout_pad, bout)
